# Initial kernel scaffold; baseline (speedup 1.0000x reference)
import numpy as np
import jax
import jax.numpy as jnp
from jax.experimental import pallas as pl

_PERM4 = tuple(
    int(x) for x in np.asarray(jax.random.permutation(jax.random.key(42), 8))[:4]
)

_TILE = 2048


def _copy_body(x_ref, o_ref):
    o_ref[...] = x_ref[...]


def kernel(inputs):
    n, rows, width = inputs.shape
    x2d = inputs.reshape(n, rows * width)
    perm_arr = jnp.asarray(_PERM4, dtype=jnp.int32)

    out2d = pl.pallas_call(
        _copy_body,
        grid=(n // _TILE, 4),
        in_specs=[
            pl.BlockSpec((_TILE, width), lambda b, j: (b, perm_arr[j])),
        ],
        out_specs=pl.BlockSpec((_TILE, width), lambda b, j: (b, j)),
        out_shape=jax.ShapeDtypeStruct((n, 4 * width), inputs.dtype),
    )(x2d)
    return out2d.reshape(n, 1, 4 * width)


# speedup vs baseline: 1.1765x; 1.1765x over previous
import numpy as np
import jax
import jax.numpy as jnp
from jax.experimental import pallas as pl

_PERM4 = tuple(
    int(x) for x in np.asarray(jax.random.permutation(jax.random.key(42), 8))[:4]
)

_TILE = 2048


def _copy_body(x0, x1, x2, x3, o_ref):
    o_ref[:, 0:128] = x0[...]
    o_ref[:, 128:256] = x1[...]
    o_ref[:, 256:384] = x2[...]
    o_ref[:, 384:512] = x3[...]


def kernel(inputs):
    n, rows, width = inputs.shape
    x2d = inputs.reshape(n, rows * width)

    def in_spec(k):
        p = _PERM4[k]
        return pl.BlockSpec((_TILE, width), lambda b: (b, p))

    out2d = pl.pallas_call(
        _copy_body,
        grid=(n // _TILE,),
        in_specs=[in_spec(0), in_spec(1), in_spec(2), in_spec(3)],
        out_specs=pl.BlockSpec((_TILE, 4 * width), lambda b: (b, 0)),
        out_shape=jax.ShapeDtypeStruct((n, 4 * width), inputs.dtype),
    )(x2d, x2d, x2d, x2d)
    return out2d.reshape(n, 1, 4 * width)


# speedup vs baseline: 3.1494x; 2.6770x over previous
import functools
import jax
import jax.numpy as jnp
from jax import lax
from jax.experimental import pallas as pl
from jax.experimental.pallas import tpu as pltpu
from jax.experimental.pallas import tpu_sc as plsc

_PERM4 = (7, 4, 2, 5)

_NC = 2
_NS = 16
_NW = _NC * _NS
_CHUNK = 64


def _make_sc_kernel(n):
    rows_per_w = n // _NW
    n_chunks = rows_per_w // _CHUNK
    mesh = plsc.VectorSubcoreMesh(core_axis_name="c", subcore_axis_name="s")

    @functools.partial(
        pl.kernel,
        mesh=mesh,
        out_type=jax.ShapeDtypeStruct((n, 4, 128), jnp.float32),
        scratch_types=[
            pltpu.VMEM((2, _CHUNK, 4, 128), jnp.float32),
            pltpu.SemaphoreType.DMA,
            pltpu.SemaphoreType.DMA,
        ],
    )
    def sc_select(x_hbm, out_hbm, buf, sem_in, sem_out):
        wid = lax.axis_index("s") * _NC + lax.axis_index("c")
        base = wid * rows_per_w

        def fire_gathers(ci, b):
            row0 = base + ci * _CHUNK
            return [
                pltpu.async_copy(
                    x_hbm.at[pl.ds(row0, _CHUNK), pl.ds(p, 1), :],
                    buf.at[b, :, pl.ds(k, 1), :],
                    sem_in,
                )
                for k, p in enumerate(_PERM4)
            ]

        in_cps = [None] * n_chunks
        out_cps = [None] * n_chunks
        in_cps[0] = fire_gathers(0, 0)
        for ci in range(n_chunks):
            b = ci % 2
            if ci + 1 < n_chunks:
                if ci >= 1:
                    out_cps[ci - 1].wait()
                in_cps[ci + 1] = fire_gathers(ci + 1, 1 - b)
            for cp in in_cps[ci]:
                cp.wait()
            row0 = base + ci * _CHUNK
            out_cps[ci] = pltpu.async_copy(
                buf.at[b], out_hbm.at[pl.ds(row0, _CHUNK)], sem_out
            )
        out_cps[n_chunks - 2].wait()
        out_cps[n_chunks - 1].wait()

    return sc_select


def kernel(inputs):
    n = inputs.shape[0]
    out = _make_sc_kernel(n)(inputs)
    return out.reshape(n, 1, 512)


# speedup vs baseline: 3.1573x; 1.0025x over previous
import functools
import jax
import jax.numpy as jnp
from jax import lax
from jax.experimental import pallas as pl
from jax.experimental.pallas import tpu as pltpu
from jax.experimental.pallas import tpu_sc as plsc

_PERM4 = (7, 4, 2, 5)

_NC = 2
_NS = 16
_NW = _NC * _NS
_CHUNK = 64


def _make_sc_kernel(n):
    rows_per_w = n // _NW
    n_chunks = rows_per_w // _CHUNK
    mesh = plsc.VectorSubcoreMesh(core_axis_name="c", subcore_axis_name="s")

    @functools.partial(
        pl.kernel,
        mesh=mesh,
        compiler_params=pltpu.CompilerParams(use_tc_tiling_on_sc=True),
        out_type=jax.ShapeDtypeStruct((n, 4, 128), jnp.float32),
        scratch_types=[
            pltpu.VMEM((2, _CHUNK, 4, 128), jnp.float32),
            pltpu.SemaphoreType.DMA,
            pltpu.SemaphoreType.DMA,
        ],
    )
    def sc_select(x_hbm, out_hbm, buf, sem_in, sem_out):
        wid = lax.axis_index("s") * _NC + lax.axis_index("c")
        base = wid * rows_per_w

        def fire_gathers(ci, b):
            row0 = base + ci * _CHUNK
            return [
                pltpu.async_copy(
                    x_hbm.at[pl.ds(row0, _CHUNK), pl.ds(p, 1), :],
                    buf.at[b, :, pl.ds(k, 1), :],
                    sem_in,
                )
                for k, p in enumerate(_PERM4)
            ]

        in_cps = [None] * n_chunks
        out_cps = [None] * n_chunks
        in_cps[0] = fire_gathers(0, 0)
        for ci in range(n_chunks):
            b = ci % 2
            if ci + 1 < n_chunks:
                if ci >= 1:
                    out_cps[ci - 1].wait()
                in_cps[ci + 1] = fire_gathers(ci + 1, 1 - b)
            for cp in in_cps[ci]:
                cp.wait()
            row0 = base + ci * _CHUNK
            out_cps[ci] = pltpu.async_copy(
                buf.at[b], out_hbm.at[pl.ds(row0, _CHUNK)], sem_out
            )
        out_cps[n_chunks - 2].wait()
        out_cps[n_chunks - 1].wait()

    return sc_select


def kernel(inputs):
    n = inputs.shape[0]
    out = _make_sc_kernel(n)(inputs)
    return out.reshape(n, 1, 512)


# speedup vs baseline: 3.2020x; 1.0141x over previous
import functools
import jax
import jax.numpy as jnp
from jax import lax
from jax.experimental import pallas as pl
from jax.experimental.pallas import tpu as pltpu
from jax.experimental.pallas import tpu_sc as plsc

_PERM4 = (7, 4, 2, 5)

_NC = 2
_NS = 16
_NW = _NC * _NS
_CHUNK = 64


def _make_sc_kernel(n):
    rows_per_w = n // _NW
    n_chunks = rows_per_w // _CHUNK
    mesh = plsc.VectorSubcoreMesh(core_axis_name="c", subcore_axis_name="s")

    @functools.partial(
        pl.kernel,
        mesh=mesh,
        compiler_params=pltpu.CompilerParams(use_tc_tiling_on_sc=True),
        out_type=jax.ShapeDtypeStruct((n, 4, 128), jnp.float32),
        scratch_types=[
            pltpu.VMEM((2, _CHUNK, 4, 128), jnp.float32),
            pltpu.SemaphoreType.DMA,
            pltpu.SemaphoreType.DMA,
        ],
    )
    def sc_select(x_hbm, out_hbm, buf, sem_in, sem_out):
        wid = lax.axis_index("s") * _NC + lax.axis_index("c")
        base = wid * rows_per_w

        def fire_gathers(ci, b):
            row0 = base + ci * _CHUNK
            for k, p in enumerate(_PERM4):
                pltpu.async_copy(
                    x_hbm.at[pl.ds(row0, _CHUNK), pl.ds(p, 1), :],
                    buf.at[b, :, pl.ds(k, 1), :],
                    sem_in,
                )

        def wait_gathers():
            for k in range(4):
                pltpu.make_async_copy(
                    x_hbm.at[pl.ds(0, _CHUNK), pl.ds(0, 1), :],
                    buf.at[0, :, pl.ds(0, 1), :],
                    sem_in,
                ).wait()

        def fire_scatter(ci, b):
            row0 = base + ci * _CHUNK
            pltpu.async_copy(buf.at[b], out_hbm.at[pl.ds(row0, _CHUNK)], sem_out)

        def wait_scatter():
            pltpu.make_async_copy(
                buf.at[0], out_hbm.at[pl.ds(0, _CHUNK)], sem_out
            ).wait()

        def half_step(ci, b):
            @pl.when(jnp.logical_and(ci >= 1, ci + 1 < n_chunks))
            def _():
                wait_scatter()

            @pl.when(ci + 1 < n_chunks)
            def _():
                fire_gathers(ci + 1, 1 - b)

            wait_gathers()
            fire_scatter(ci, b)

        fire_gathers(0, 0)

        def pair_body(i, carry):
            half_step(2 * i, 0)
            half_step(2 * i + 1, 1)
            return carry

        lax.fori_loop(0, n_chunks // 2, pair_body, 0)
        wait_scatter()
        wait_scatter()

    return sc_select


def kernel(inputs):
    n = inputs.shape[0]
    out = _make_sc_kernel(n)(inputs)
    return out.reshape(n, 1, 512)


# speedup vs baseline: 3.2268x; 1.0078x over previous
import functools
import jax
import jax.numpy as jnp
from jax import lax
from jax.experimental import pallas as pl
from jax.experimental.pallas import tpu as pltpu
from jax.experimental.pallas import tpu_sc as plsc

_PERM4 = (7, 4, 2, 5)

_NC = 2
_NS = 16
_NW = _NC * _NS
_CHUNK = 64
_NBUF = 3


def _make_sc_kernel(n):
    rows_per_w = n // _NW
    n_chunks = rows_per_w // _CHUNK
    mesh = plsc.VectorSubcoreMesh(core_axis_name="c", subcore_axis_name="s")

    @functools.partial(
        pl.kernel,
        mesh=mesh,
        compiler_params=pltpu.CompilerParams(use_tc_tiling_on_sc=True),
        out_type=jax.ShapeDtypeStruct((n, 4, 128), jnp.float32),
        scratch_types=[
            pltpu.VMEM((_NBUF, _CHUNK, 4, 128), jnp.float32),
            pltpu.SemaphoreType.DMA,
            pltpu.SemaphoreType.DMA,
        ],
    )
    def sc_select(x_hbm, out_hbm, buf, sem_in, sem_out):
        wid = lax.axis_index("s") * _NC + lax.axis_index("c")
        base = wid * rows_per_w

        def fire_gathers(ci, b):
            row0 = base + ci * _CHUNK
            for k, p in enumerate(_PERM4):
                pltpu.async_copy(
                    x_hbm.at[pl.ds(row0, _CHUNK), pl.ds(p, 1), :],
                    buf.at[b, :, pl.ds(k, 1), :],
                    sem_in,
                )

        def wait_gathers():
            for _ in range(4):
                pltpu.make_async_copy(
                    x_hbm.at[pl.ds(0, _CHUNK), pl.ds(0, 1), :],
                    buf.at[0, :, pl.ds(0, 1), :],
                    sem_in,
                ).wait()

        def fire_scatter(ci, b):
            row0 = base + ci * _CHUNK
            pltpu.async_copy(buf.at[b], out_hbm.at[pl.ds(row0, _CHUNK)], sem_out)

        def wait_scatter():
            pltpu.make_async_copy(
                buf.at[0], out_hbm.at[pl.ds(0, _CHUNK)], sem_out
            ).wait()

        fire_gathers(0, 0)
        fire_gathers(1, 1)
        for ci in range(n_chunks):
            b = ci % _NBUF
            if ci + 2 < n_chunks:
                if ci >= 1:
                    wait_scatter()
                fire_gathers(ci + 2, (ci + 2) % _NBUF)
            wait_gathers()
            fire_scatter(ci, b)
        for _ in range(3):
            wait_scatter()

    return sc_select


def kernel(inputs):
    n = inputs.shape[0]
    out = _make_sc_kernel(n)(inputs)
    return out.reshape(n, 1, 512)


# speedup vs baseline: 3.2333x; 1.0020x over previous
import functools
import jax
import jax.numpy as jnp
from jax import lax
from jax.experimental import pallas as pl
from jax.experimental.pallas import tpu as pltpu
from jax.experimental.pallas import tpu_sc as plsc

_PERM4 = (7, 4, 2, 5)

_NC = 2
_NS = 16
_NW = _NC * _NS
_CHUNK = 64
_NBUF = 3


def _make_sc_kernel(n):
    rows_per_w = n // _NW
    n_chunks = rows_per_w // _CHUNK
    mesh = plsc.VectorSubcoreMesh(core_axis_name="c", subcore_axis_name="s")

    @functools.partial(
        pl.kernel,
        mesh=mesh,
        compiler_params=pltpu.CompilerParams(
            use_tc_tiling_on_sc=True,
            disable_bounds_checks=True,
            disable_semaphore_checks=True,
            skip_device_barrier=True,
        ),
        out_type=jax.ShapeDtypeStruct((n, 4, 128), jnp.float32),
        scratch_types=[
            pltpu.VMEM((_NBUF, _CHUNK, 4, 128), jnp.float32),
            pltpu.SemaphoreType.DMA,
            pltpu.SemaphoreType.DMA,
        ],
    )
    def sc_select(x_hbm, out_hbm, buf, sem_in, sem_out):
        wid = lax.axis_index("s") * _NC + lax.axis_index("c")
        base = wid * rows_per_w

        def fire_gathers(ci, b):
            row0 = base + ci * _CHUNK
            for k, p in enumerate(_PERM4):
                pltpu.async_copy(
                    x_hbm.at[pl.ds(row0, _CHUNK), pl.ds(p, 1), :],
                    buf.at[b, :, pl.ds(k, 1), :],
                    sem_in,
                )

        def wait_gathers():
            for _ in range(4):
                pltpu.make_async_copy(
                    x_hbm.at[pl.ds(0, _CHUNK), pl.ds(0, 1), :],
                    buf.at[0, :, pl.ds(0, 1), :],
                    sem_in,
                ).wait()

        def fire_scatter(ci, b):
            row0 = base + ci * _CHUNK
            pltpu.async_copy(buf.at[b], out_hbm.at[pl.ds(row0, _CHUNK)], sem_out)

        def wait_scatter():
            pltpu.make_async_copy(
                buf.at[0], out_hbm.at[pl.ds(0, _CHUNK)], sem_out
            ).wait()

        fire_gathers(0, 0)
        fire_gathers(1, 1)
        for ci in range(n_chunks):
            b = ci % _NBUF
            if ci + 2 < n_chunks:
                if ci >= 1:
                    wait_scatter()
                fire_gathers(ci + 2, (ci + 2) % _NBUF)
            wait_gathers()
            fire_scatter(ci, b)
        for _ in range(3):
            wait_scatter()

    return sc_select


def kernel(inputs):
    n = inputs.shape[0]
    out = _make_sc_kernel(n)(inputs)
    return out.reshape(n, 1, 512)
